# Initial kernel scaffold; baseline (speedup 1.0000x reference)
import functools

import jax
import jax.numpy as jnp
from jax import lax
from jax.experimental import pallas as pl
from jax.experimental.pallas import tpu as pltpu
from jax.experimental.pallas import tpu_sc as plsc

NC = 2
NS = 16
NW = NC * NS
L = 16

f32 = jnp.float32
i32 = jnp.int32


def _mesh():
    return plsc.VectorSubcoreMesh(
        core_axis_name="c", subcore_axis_name="s", num_cores=NC, num_subcores=NS
    )


def _leaky(v):
    return jnp.where(v >= 0.0, v, 0.2 * v)


def _emb_gather(x_pad, emb, npad):
    V, D = emb.shape
    per_w = npad // NW
    chunks = []
    off = 0
    while off < per_w:
        c = min(128, per_w - off)
        chunks.append((off, c))
        off += c

    @functools.partial(
        pl.kernel,
        out_type=jax.ShapeDtypeStruct((npad, D), f32),
        mesh=_mesh(),
        scratch_types=[
            pltpu.VMEM((per_w,), i32),
            pltpu.VMEM((128, D), f32),
            pltpu.SemaphoreType.DMA,
        ],
    )
    def k(x_h, emb_h, h_h, idx_v, rows_v, sem):
        wid = lax.axis_index("s") * NC + lax.axis_index("c")
        base = wid * per_w
        pltpu.sync_copy(x_h.at[pl.ds(base, per_w)], idx_v)
        for off, cnt in chunks:
            pltpu.async_copy(
                emb_h.at[idx_v.at[pl.ds(off, cnt)]], rows_v.at[pl.ds(0, cnt)], sem
            ).wait()
            pltpu.sync_copy(rows_v.at[pl.ds(0, cnt)], h_h.at[pl.ds(base + off, cnt)])

    return k(x_pad, emb)


def _tc_proj(h, Wg, a_s, a_d):
    npad, D = h.shape
    BLK = 2048
    G = npad // BLK

    def body(h_ref, wg_ref, as_ref, ad_ref, hp_ref, al_ref, ar_ref, c_ref, mx_ref):
        i = pl.program_id(0)
        hp = jnp.dot(h_ref[...], wg_ref[...], preferred_element_type=f32)
        hp_ref[...] = hp
        al = jnp.sum(hp * as_ref[...][None, :], axis=1)
        ar = jnp.sum(hp * ad_ref[...][None, :], axis=1)
        al_ref[...] = al
        ar_ref[...] = ar

        @pl.when(i == 0)
        def _():
            mx_ref[0] = -jnp.inf
            mx_ref[1] = -jnp.inf

        mx_ref[0] = jnp.maximum(mx_ref[0], jnp.max(al))
        mx_ref[1] = jnp.maximum(mx_ref[1], jnp.max(ar))

        @pl.when(i == G - 1)
        def _():
            m = mx_ref[0] + mx_ref[1]
            c = jnp.where(m >= 0.0, m, 0.2 * m)
            c_ref[...] = jnp.full((1, L), c, f32)

    return pl.pallas_call(
        body,
        grid=(G,),
        in_specs=[
            pl.BlockSpec((BLK, D), lambda i: (i, 0)),
            pl.BlockSpec((D, D), lambda i: (0, 0)),
            pl.BlockSpec((D,), lambda i: (0,)),
            pl.BlockSpec((D,), lambda i: (0,)),
        ],
        out_specs=[
            pl.BlockSpec((BLK, D), lambda i: (i, 0)),
            pl.BlockSpec((BLK,), lambda i: (i,)),
            pl.BlockSpec((BLK,), lambda i: (i,)),
            pl.BlockSpec((1, L), lambda i: (0, 0)),
        ],
        out_shape=[
            jax.ShapeDtypeStruct((npad, D), f32),
            jax.ShapeDtypeStruct((npad,), f32),
            jax.ShapeDtypeStruct((npad,), f32),
            jax.ShapeDtypeStruct((1, L), f32),
        ],
        scratch_shapes=[pltpu.SMEM((2,), f32)],
    )(h, Wg, a_s, a_d)


def _sc_edge_softmax_denom(src, dst, al, ar, c, ept, npad):
    etpad = ept * NW
    DR = npad // 16

    @functools.partial(
        pl.kernel,
        out_type=[
            jax.ShapeDtypeStruct((etpad,), f32),
            jax.ShapeDtypeStruct((NC, npad), f32),
        ],
        mesh=_mesh(),
        scratch_types=[
            pltpu.VMEM((npad,), f32),
            pltpu.VMEM((npad,), f32),
            pltpu.VMEM((ept,), i32),
            pltpu.VMEM((ept,), i32),
            pltpu.VMEM((ept,), f32),
            pltpu.VMEM((16, npad // 16), f32),
            pltpu.VMEM((L,), f32),
            pltpu.VMEM((L,), i32),
            pltpu.VMEM_SHARED((16, npad // 16), f32),
        ],
    )
    def k(src_h, dst_h, al_h, ar_h, c_h, ex_h, denp_h, al_v, ar_v, s_v, d_v,
          ex_v, den_v, c_v, idx_v, den_sh):
        core = lax.axis_index("c")
        sub = lax.axis_index("s")
        wid = sub * NC + core
        base = wid * ept
        pltpu.sync_copy(al_h, al_v)
        pltpu.sync_copy(ar_h, ar_v)
        pltpu.sync_copy(c_h.at[0], c_v)
        pltpu.sync_copy(src_h.at[pl.ds(base, ept)], s_v)
        pltpu.sync_copy(dst_h.at[pl.ds(base, ept)], d_v)
        idx_v[...] = lax.iota(i32, L)

        zero = jnp.zeros((L,), f32)

        def zrow(j, _):
            def zcol(q, _):
                den_v[j, pl.ds(q * L, L)] = zero
                return ()
            lax.fori_loop(0, DR // L, zcol, ())
            return ()

        lax.fori_loop(0, 16, zrow, ())
        pltpu.sync_copy(den_v.at[0], den_sh.at[sub])
        plsc.subcore_barrier()

        c16 = c_v[...]

        def ebody(e, _):
            s16 = s_v[pl.ds(e * L, L)]
            d16 = d_v[pl.ds(e * L, L)]
            a = plsc.load_gather(al_v, [s16])
            b = plsc.load_gather(ar_v, [d16])
            ex = jnp.exp(_leaky(a + b) - c16)
            ex_v[pl.ds(e * L, L)] = ex
            plsc.addupdate_scatter(den_v, [d16 // DR, d16 % DR], ex)
            return ()

        lax.fori_loop(0, ept // L, ebody, ())
        pltpu.sync_copy(ex_v, ex_h.at[pl.ds(base, ept)])
        pltpu.sync_copy(den_v, den_sh.at[idx_v], add=True)
        plsc.subcore_barrier()
        pltpu.sync_copy(den_sh.at[sub], denp_h.at[core, pl.ds(sub * DR, DR)])

    return k(src, dst, al, ar, c)


def _sc_aggregate(src, dstb, ex, denp, hp, ept, npad):
    D = hp.shape[1]
    NBLK = ept // 128
    RPT = npad // 16

    @functools.partial(
        pl.kernel,
        out_type=[
            jax.ShapeDtypeStruct((npad, D), f32),
            jax.ShapeDtypeStruct((npad, D), f32),
        ],
        mesh=_mesh(),
        scratch_types=[
            pltpu.VMEM((npad,), f32),
            pltpu.VMEM((npad,), f32),
            pltpu.VMEM((ept,), i32),
            pltpu.VMEM((NBLK, 128), i32),
            pltpu.VMEM((ept,), f32),
            pltpu.VMEM((128, D), f32),
            pltpu.SemaphoreType.DMA,
            pltpu.VMEM_SHARED((npad, D), f32),
        ],
    )
    def k(src_h, dstb_h, ex_h, denp_h, hp_h, p0_h, p1_h, den_v, tmp_v, s_v,
          db_v, a_v, rows_v, sem, out_sh):
        core = lax.axis_index("c")
        sub = lax.axis_index("s")
        wid = sub * NC + core
        base = wid * ept
        pltpu.sync_copy(denp_h.at[0], den_v)
        pltpu.sync_copy(denp_h.at[1], tmp_v)
        pltpu.sync_copy(src_h.at[pl.ds(base, ept)], s_v)
        pltpu.sync_copy(dstb_h.at[wid], db_v)
        pltpu.sync_copy(ex_h.at[pl.ds(base, ept)], a_v)

        def dbody(j, _):
            sl = pl.ds(j * L, L)
            den_v[sl] = den_v[sl] + tmp_v[sl]
            return ()

        lax.fori_loop(0, npad // L, dbody, ())

        def zrow(r, _):
            for q in range(D // L):
                rows_v[r, pl.ds(q * L, L)] = jnp.zeros((L,), f32)
            return ()

        lax.fori_loop(0, 128, zrow, ())
        for j in range(RPT // 128):
            pltpu.sync_copy(rows_v, out_sh.at[pl.ds(sub * RPT + j * 128, 128)])
        plsc.subcore_barrier()

        def abody(e, _):
            b = e // (128 // L)
            g = e % (128 // L)
            d16 = db_v[b, pl.ds(g * L, L)]
            dn = plsc.load_gather(den_v, [d16])
            sl = pl.ds(e * L, L)
            a_v[sl] = a_v[sl] / (dn + 1e-16)
            return ()

        lax.fori_loop(0, ept // L, abody, ())

        def gbody(b, _):
            pltpu.async_copy(
                hp_h.at[s_v.at[pl.ds(b * 128, 128)]], rows_v, sem
            ).wait()

            def sbody(g, _):
                a16 = a_v[pl.ds(b * 128 + g * L, L)]
                for r in range(L):
                    av = jnp.take(a16, jnp.full((L,), r, i32),
                                  mode="promise_in_bounds")
                    row = g * L + r
                    for q in range(D // L):
                        sl = pl.ds(q * L, L)
                        rows_v[row, sl] = rows_v[row, sl] * av
                return ()

            lax.fori_loop(0, 128 // L, sbody, ())
            pltpu.sync_copy(rows_v, out_sh.at[db_v.at[b]], add=True)
            return ()

        lax.fori_loop(0, NBLK, gbody, ())
        plsc.subcore_barrier()

        @pl.when(core == 0)
        def _():
            for j in range(RPT // 128):
                sl = pl.ds(sub * RPT + j * 128, 128)
                pltpu.sync_copy(out_sh.at[sl], rows_v)
                pltpu.sync_copy(rows_v, p0_h.at[sl])

        @pl.when(core == 1)
        def _():
            for j in range(RPT // 128):
                sl = pl.ds(sub * RPT + j * 128, 128)
                pltpu.sync_copy(out_sh.at[sl], rows_v)
                pltpu.sync_copy(rows_v, p1_h.at[sl])

    return k(src, dstb, ex, denp, hp)


def _tc_mid(p0, p1, bg0, Wl0, bl0, Wg1, as1, ad1):
    npad, D = p0.shape
    BLK = 2048
    G = npad // BLK

    def body(p0_ref, p1_ref, bg_ref, wl_ref, bl_ref, wg_ref, as_ref, ad_ref,
             hp_ref, al_ref, ar_ref, c_ref, mx_ref):
        i = pl.program_id(0)
        o = p0_ref[...] + p1_ref[...] + bg_ref[...][None, :]
        o = jnp.maximum(o, 0.0)
        h1 = jnp.dot(o, wl_ref[...], preferred_element_type=f32) + bl_ref[...][None, :]
        hp = jnp.dot(h1, wg_ref[...], preferred_element_type=f32)
        hp_ref[...] = hp
        al = jnp.sum(hp * as_ref[...][None, :], axis=1)
        ar = jnp.sum(hp * ad_ref[...][None, :], axis=1)
        al_ref[...] = al
        ar_ref[...] = ar

        @pl.when(i == 0)
        def _():
            mx_ref[0] = -jnp.inf
            mx_ref[1] = -jnp.inf

        mx_ref[0] = jnp.maximum(mx_ref[0], jnp.max(al))
        mx_ref[1] = jnp.maximum(mx_ref[1], jnp.max(ar))

        @pl.when(i == G - 1)
        def _():
            m = mx_ref[0] + mx_ref[1]
            c = jnp.where(m >= 0.0, m, 0.2 * m)
            c_ref[...] = jnp.full((1, L), c, f32)

    return pl.pallas_call(
        body,
        grid=(G,),
        in_specs=[
            pl.BlockSpec((BLK, D), lambda i: (i, 0)),
            pl.BlockSpec((BLK, D), lambda i: (i, 0)),
            pl.BlockSpec((D,), lambda i: (0,)),
            pl.BlockSpec((D, D), lambda i: (0, 0)),
            pl.BlockSpec((D,), lambda i: (0,)),
            pl.BlockSpec((D, D), lambda i: (0, 0)),
            pl.BlockSpec((D,), lambda i: (0,)),
            pl.BlockSpec((D,), lambda i: (0,)),
        ],
        out_specs=[
            pl.BlockSpec((BLK, D), lambda i: (i, 0)),
            pl.BlockSpec((BLK,), lambda i: (i,)),
            pl.BlockSpec((BLK,), lambda i: (i,)),
            pl.BlockSpec((1, L), lambda i: (0, 0)),
        ],
        out_shape=[
            jax.ShapeDtypeStruct((npad, D), f32),
            jax.ShapeDtypeStruct((npad,), f32),
            jax.ShapeDtypeStruct((npad,), f32),
            jax.ShapeDtypeStruct((1, L), f32),
        ],
        scratch_shapes=[pltpu.SMEM((2,), f32)],
    )(p0, p1, bg0, Wl0, bl0, Wg1, as1, ad1)


def _tc_pool(p0, p1, bg1, Wl1, bl1, n_real):
    npad, D = p0.shape
    BLK = 2048
    G = npad // BLK

    def body(p0_ref, p1_ref, bg_ref, wl_ref, bl_ref, out_ref, acc_ref):
        i = pl.program_id(0)
        o = p0_ref[...] + p1_ref[...] + bg_ref[...][None, :]
        o = jnp.maximum(o, 0.0)
        rows = lax.broadcasted_iota(i32, (BLK, 1), 0) + i * BLK
        o = jnp.where(rows < n_real, o, 0.0)
        s = jnp.sum(o, axis=0, keepdims=True)

        @pl.when(i == 0)
        def _():
            acc_ref[...] = jnp.zeros((1, D), f32)

        acc_ref[...] = acc_ref[...] + s

        @pl.when(i == G - 1)
        def _():
            pooled = acc_ref[...] * (1.0 / n_real)
            out_ref[...] = (
                jnp.dot(pooled, wl_ref[...], preferred_element_type=f32)
                + bl_ref[...][None, :]
            )

    return pl.pallas_call(
        body,
        grid=(G,),
        in_specs=[
            pl.BlockSpec((BLK, D), lambda i: (i, 0)),
            pl.BlockSpec((BLK, D), lambda i: (i, 0)),
            pl.BlockSpec((D,), lambda i: (0,)),
            pl.BlockSpec((D, D), lambda i: (0, 0)),
            pl.BlockSpec((D,), lambda i: (0,)),
        ],
        out_specs=pl.BlockSpec((1, D), lambda i: (0, 0)),
        out_shape=jax.ShapeDtypeStruct((1, D), f32),
        scratch_shapes=[pltpu.VMEM((1, D), f32)],
    )(p0, p1, bg1, Wl1, bl1)


def _tc_out(pooled, Wout, bout2):
    D, V = Wout.shape
    BV = 8192
    G = pl.cdiv(V, BV)

    def body(p_ref, w_ref, b_ref, o_ref):
        o_ref[...] = (
            jnp.dot(p_ref[...], w_ref[...], preferred_element_type=f32) + b_ref[...]
        )

    return pl.pallas_call(
        body,
        grid=(G,),
        in_specs=[
            pl.BlockSpec((1, D), lambda i: (0, 0)),
            pl.BlockSpec((D, BV), lambda i: (0, i)),
            pl.BlockSpec((1, BV), lambda i: (0, i)),
        ],
        out_specs=pl.BlockSpec((1, BV), lambda i: (0, i)),
        out_shape=jax.ShapeDtypeStruct((1, V), f32),
    )(pooled, Wout, bout2)


def kernel(x, edge_index, emb, Wg0, as0, ad0, bg0, Wl0, bl0,
           Wg1, as1, ad1, bg1, Wl1, bl1, Wout, bout):
    N = x.shape[0]
    V, D = emb.shape
    E = edge_index.shape[1]

    npad = ((N + 1 + 2047) // 2048) * 2048
    et = E + N
    ept = ((et + NW - 1) // NW + 127) // 128 * 128
    etpad = ept * NW

    x_pad = jnp.concatenate([x.astype(i32), jnp.zeros((npad - N,), i32)])
    loops = jnp.arange(N, dtype=i32)
    padi = jnp.full((etpad - et,), N, i32)
    src = jnp.concatenate([edge_index[0].astype(i32), loops, padi])
    dst = jnp.concatenate([edge_index[1].astype(i32), loops, padi])
    dstb = dst.reshape(NW, ept // 128, 128)
    bout2 = bout.reshape(1, V)

    h = _emb_gather(x_pad, emb, npad)

    hp0, al0, ar0, c0 = _tc_proj(h, Wg0, as0, ad0)
    ex0, denp0 = _sc_edge_softmax_denom(src, dst, al0, ar0, c0, ept, npad)
    p00, p01 = _sc_aggregate(src, dstb, ex0, denp0, hp0, ept, npad)

    hp1, al1, ar1, c1 = _tc_mid(p00, p01, bg0, Wl0, bl0, Wg1, as1, ad1)
    ex1, denp1 = _sc_edge_softmax_denom(src, dst, al1, ar1, c1, ept, npad)
    p10, p11 = _sc_aggregate(src, dstb, ex1, denp1, hp1, ept, npad)

    pooled = _tc_pool(p10, p11, bg1, Wl1, bl1, N)
    return _tc_out(pooled, Wout, bout2)


# speedup vs baseline: 31.2125x; 31.2125x over previous
import functools

import jax
import jax.numpy as jnp
from jax import lax
from jax.experimental import pallas as pl
from jax.experimental.pallas import tpu as pltpu
from jax.experimental.pallas import tpu_sc as plsc

NC = 2
NS = 16
NW = NC * NS
L = 16

f32 = jnp.float32
i32 = jnp.int32


def _mesh():
    return plsc.VectorSubcoreMesh(
        core_axis_name="c", subcore_axis_name="s", num_cores=NC, num_subcores=NS
    )


def _leaky(v):
    return jnp.where(v >= 0.0, v, 0.2 * v)


def _bcast_lane(v16, r):
    return lax.gather(
        v16,
        jnp.full((L, 1), r, i32),
        lax.GatherDimensionNumbers(
            offset_dims=(), collapsed_slice_dims=(0,), start_index_map=(0,)
        ),
        slice_sizes=(1,),
        mode=lax.GatherScatterMode.PROMISE_IN_BOUNDS,
    )


def _emb_gather(x_pad, emb, npad):
    V, D = emb.shape
    per_w = npad // NW
    chunks = []
    off = 0
    while off < per_w:
        c = min(128, per_w - off)
        chunks.append((off, c))
        off += c

    @functools.partial(
        pl.kernel,
        out_type=jax.ShapeDtypeStruct((npad, D), f32),
        mesh=_mesh(),
        compiler_params=pltpu.CompilerParams(needs_layout_passes=False),
        scratch_types=[
            pltpu.VMEM((per_w,), i32),
            pltpu.VMEM((128, D), f32),
            pltpu.SemaphoreType.DMA,
        ],
    )
    def k(x_h, emb_h, h_h, idx_v, rows_v, sem):
        wid = lax.axis_index("s") * NC + lax.axis_index("c")
        base = wid * per_w
        pltpu.sync_copy(x_h.at[pl.ds(base, per_w)], idx_v)
        for off, cnt in chunks:
            pltpu.async_copy(
                emb_h.at[idx_v.at[pl.ds(off, cnt)]], rows_v.at[pl.ds(0, cnt)], sem
            ).wait()
            pltpu.sync_copy(rows_v.at[pl.ds(0, cnt)], h_h.at[pl.ds(base + off, cnt)])

    return k(x_pad, emb)


def _tc_proj(h, Wg, a_s, a_d):
    npad, D = h.shape
    BLK = 2048
    G = npad // BLK

    def body(h_ref, wg_ref, as_ref, ad_ref, hp_ref, al_ref, ar_ref, c_ref, mx_ref):
        i = pl.program_id(0)
        hp = jnp.dot(h_ref[...], wg_ref[...], preferred_element_type=f32)
        hp_ref[...] = hp
        al = jnp.sum(hp * as_ref[...][None, :], axis=1)
        ar = jnp.sum(hp * ad_ref[...][None, :], axis=1)
        al_ref[...] = al
        ar_ref[...] = ar

        @pl.when(i == 0)
        def _():
            mx_ref[0] = -jnp.inf
            mx_ref[1] = -jnp.inf

        mx_ref[0] = jnp.maximum(mx_ref[0], jnp.max(al))
        mx_ref[1] = jnp.maximum(mx_ref[1], jnp.max(ar))

        @pl.when(i == G - 1)
        def _():
            m = mx_ref[0] + mx_ref[1]
            c = jnp.where(m >= 0.0, m, 0.2 * m)
            c_ref[...] = jnp.full((1, L), c, f32)

    return pl.pallas_call(
        body,
        grid=(G,),
        in_specs=[
            pl.BlockSpec((BLK, D), lambda i: (i, 0)),
            pl.BlockSpec((D, D), lambda i: (0, 0)),
            pl.BlockSpec((D,), lambda i: (0,)),
            pl.BlockSpec((D,), lambda i: (0,)),
        ],
        out_specs=[
            pl.BlockSpec((BLK, D), lambda i: (i, 0)),
            pl.BlockSpec((BLK,), lambda i: (i,)),
            pl.BlockSpec((BLK,), lambda i: (i,)),
            pl.BlockSpec((1, L), lambda i: (0, 0)),
        ],
        out_shape=[
            jax.ShapeDtypeStruct((npad, D), f32),
            jax.ShapeDtypeStruct((npad,), f32),
            jax.ShapeDtypeStruct((npad,), f32),
            jax.ShapeDtypeStruct((1, L), f32),
        ],
        scratch_shapes=[pltpu.SMEM((2,), f32)],
    )(h, Wg, a_s, a_d)


def _sc_edge_softmax_denom(src, dst, al, ar, c, ept, npad):
    etpad = ept * NW
    DR = npad // 16

    @functools.partial(
        pl.kernel,
        out_type=[
            jax.ShapeDtypeStruct((etpad,), f32),
            jax.ShapeDtypeStruct((NW, npad), f32),
        ],
        mesh=_mesh(),
        compiler_params=pltpu.CompilerParams(needs_layout_passes=False),
        scratch_types=[
            pltpu.VMEM((npad,), f32),
            pltpu.VMEM((npad,), f32),
            pltpu.VMEM((ept,), i32),
            pltpu.VMEM((ept,), i32),
            pltpu.VMEM((ept,), f32),
            pltpu.VMEM((npad,), f32),
            pltpu.VMEM((L,), f32),
        ],
    )
    def k(src_h, dst_h, al_h, ar_h, c_h, ex_h, denp_h, al_v, ar_v, s_v, d_v,
          ex_v, den_v, c_v):
        core = lax.axis_index("c")
        sub = lax.axis_index("s")
        wid = sub * NC + core
        base = wid * ept
        pltpu.sync_copy(al_h, al_v)
        pltpu.sync_copy(ar_h, ar_v)
        pltpu.sync_copy(c_h.at[0], c_v)
        pltpu.sync_copy(src_h.at[pl.ds(base, ept)], s_v)
        pltpu.sync_copy(dst_h.at[pl.ds(base, ept)], d_v)

        zero = jnp.zeros((L,), f32)

        def zbody(q, _):
            den_v[pl.ds(q * L, L)] = zero
            return ()

        lax.fori_loop(0, npad // L, zbody, ())

        c16 = c_v[...]

        def ebody(e, _):
            s16 = s_v[pl.ds(e * L, L)]
            d16 = d_v[pl.ds(e * L, L)]
            a = plsc.load_gather(al_v, [s16])
            b = plsc.load_gather(ar_v, [d16])
            ex = jnp.exp(_leaky(a + b) - c16)
            ex_v[pl.ds(e * L, L)] = ex
            plsc.addupdate_scatter(den_v, [d16], ex)
            return ()

        lax.fori_loop(0, ept // L, ebody, ())
        pltpu.sync_copy(ex_v, ex_h.at[pl.ds(base, ept)])
        pltpu.sync_copy(den_v, denp_h.at[wid])

    return k(src, dst, al, ar, c)


def _sc_aggregate(src, dstb, ex, hp, etpad, npad, nca, ncb):
    D = hp.shape[1]
    BPC = 9
    EC = BPC * 128
    ncht = etpad // EC
    assert 16 * (nca + ncb) == ncht
    RPT = npad // 16

    @functools.partial(
        pl.kernel,
        out_type=[
            jax.ShapeDtypeStruct((npad, D), f32),
            jax.ShapeDtypeStruct((npad, D), f32),
        ],
        mesh=_mesh(),
        compiler_params=pltpu.CompilerParams(needs_layout_passes=False),
        scratch_types=[
            pltpu.VMEM((EC,), i32),
            pltpu.VMEM((2 * BPC, 64), i32),
            pltpu.VMEM((EC,), f32),
            pltpu.VMEM((2, 128, D), f32),
            pltpu.SemaphoreType.DMA,
            pltpu.SemaphoreType.DMA,
            pltpu.SemaphoreType.DMA,
            pltpu.SemaphoreType.DMA,
            pltpu.VMEM_SHARED((npad, D), f32),
        ],
    )
    def k(src_h, dstb_h, ex_h, hp_h, p0_h, p1_h, s_v,
          db_v, a_v, rows_v, gs0, gs1, ss0, ss1, out_sh):
        core = lax.axis_index("c")
        sub = lax.axis_index("s")
        gsem = (gs0, gs1)
        ssem = (ss0, ss1)
        nch = jnp.where(core == 0, nca, ncb)
        cbase = jnp.where(core == 0, sub * nca, 16 * nca + sub * ncb)

        def zrow(r, _):
            for q in range(D // L):
                rows_v[0, r, pl.ds(q * L, L)] = jnp.zeros((L,), f32)
            return ()

        lax.fori_loop(0, 128, zrow, ())
        for j in range(RPT // 128):
            pltpu.sync_copy(rows_v.at[0], out_sh.at[pl.ds(sub * RPT + j * 128, 128)])
        plsc.subcore_barrier()

        def chunk_body(ch, _):
            base = (cbase + ch) * EC
            pltpu.sync_copy(src_h.at[pl.ds(base, EC)], s_v)
            pltpu.sync_copy(ex_h.at[pl.ds(base, EC)], a_v)
            pltpu.sync_copy(dstb_h.at[cbase + ch], db_v)

            def issue_gather(b, buf):
                return pltpu.async_copy(
                    hp_h.at[s_v.at[pl.ds(b * 128, 128)]], rows_v.at[buf],
                    gsem[buf],
                )

            def scale_half(buf, b, h):
                def sbody(i, _):
                    a16 = a_v[pl.ds(b * 128 + h * 64 + (i // L) * L, L)]
                    av = _bcast_lane(a16, i % L)
                    row = h * 64 + i
                    for q in range(D // L):
                        sl = pl.ds(q * L, L)
                        rows_v[buf, row, sl] = rows_v[buf, row, sl] * av
                    return ()

                lax.fori_loop(0, 64, sbody, ())

            gd = [None, None]
            gd[0] = issue_gather(0, 0)
            for b in range(BPC):
                buf = b % 2
                obuf = 1 - buf
                gd[buf].wait()
                if b + 1 < BPC:
                    gd[obuf] = issue_gather(b + 1, obuf)
                scale_half(buf, b, 0)
                s0 = pltpu.async_copy(
                    rows_v.at[buf, pl.ds(0, 64)], out_sh.at[db_v.at[2 * b]],
                    ssem[buf], add=True,
                )
                scale_half(buf, b, 1)
                s1 = pltpu.async_copy(
                    rows_v.at[buf, pl.ds(64, 64)], out_sh.at[db_v.at[2 * b + 1]],
                    ssem[buf], add=True,
                )
                s0.wait()
                s1.wait()
            return ()

        lax.fori_loop(0, nch, chunk_body, ())
        plsc.subcore_barrier()

        @pl.when(core == 0)
        def _():
            for j in range(RPT // 128):
                sl = pl.ds(sub * RPT + j * 128, 128)
                pltpu.sync_copy(out_sh.at[sl], rows_v.at[0])
                pltpu.sync_copy(rows_v.at[0], p0_h.at[sl])

        @pl.when(core == 1)
        def _():
            for j in range(RPT // 128):
                sl = pl.ds(sub * RPT + j * 128, 128)
                pltpu.sync_copy(out_sh.at[sl], rows_v.at[0])
                pltpu.sync_copy(rows_v.at[0], p1_h.at[sl])

    return k(src, dstb, ex, hp)


_K2_NCA = 11
_K2_NCB = 7


def _tc_mid(p0, p1, denp, bg0, Wl0, bl0, Wg1, as1, ad1):
    npad, D = p0.shape
    BLK = 2048
    G = npad // BLK

    def body(p0_ref, p1_ref, dn_ref, bg_ref, wl_ref, bl_ref, wg_ref, as_ref,
             ad_ref, hp_ref, al_ref, ar_ref, c_ref, mx_ref):
        i = pl.program_id(0)
        inv = 1.0 / (jnp.sum(dn_ref[...], axis=0) + 1e-16)
        o = (p0_ref[...] + p1_ref[...]) * inv[:, None] + bg_ref[...][None, :]
        o = jnp.maximum(o, 0.0)
        h1 = jnp.dot(o, wl_ref[...], preferred_element_type=f32) + bl_ref[...][None, :]
        hp = jnp.dot(h1, wg_ref[...], preferred_element_type=f32)
        hp_ref[...] = hp
        al = jnp.sum(hp * as_ref[...][None, :], axis=1)
        ar = jnp.sum(hp * ad_ref[...][None, :], axis=1)
        al_ref[...] = al
        ar_ref[...] = ar

        @pl.when(i == 0)
        def _():
            mx_ref[0] = -jnp.inf
            mx_ref[1] = -jnp.inf

        mx_ref[0] = jnp.maximum(mx_ref[0], jnp.max(al))
        mx_ref[1] = jnp.maximum(mx_ref[1], jnp.max(ar))

        @pl.when(i == G - 1)
        def _():
            m = mx_ref[0] + mx_ref[1]
            c = jnp.where(m >= 0.0, m, 0.2 * m)
            c_ref[...] = jnp.full((1, L), c, f32)

    return pl.pallas_call(
        body,
        grid=(G,),
        in_specs=[
            pl.BlockSpec((BLK, D), lambda i: (i, 0)),
            pl.BlockSpec((BLK, D), lambda i: (i, 0)),
            pl.BlockSpec((NW, BLK), lambda i: (0, i)),
            pl.BlockSpec((D,), lambda i: (0,)),
            pl.BlockSpec((D, D), lambda i: (0, 0)),
            pl.BlockSpec((D,), lambda i: (0,)),
            pl.BlockSpec((D, D), lambda i: (0, 0)),
            pl.BlockSpec((D,), lambda i: (0,)),
            pl.BlockSpec((D,), lambda i: (0,)),
        ],
        out_specs=[
            pl.BlockSpec((BLK, D), lambda i: (i, 0)),
            pl.BlockSpec((BLK,), lambda i: (i,)),
            pl.BlockSpec((BLK,), lambda i: (i,)),
            pl.BlockSpec((1, L), lambda i: (0, 0)),
        ],
        out_shape=[
            jax.ShapeDtypeStruct((npad, D), f32),
            jax.ShapeDtypeStruct((npad,), f32),
            jax.ShapeDtypeStruct((npad,), f32),
            jax.ShapeDtypeStruct((1, L), f32),
        ],
        scratch_shapes=[pltpu.SMEM((2,), f32)],
    )(p0, p1, denp, bg0, Wl0, bl0, Wg1, as1, ad1)


def _tc_pool(p0, p1, denp, bg1, Wl1, bl1, n_real):
    npad, D = p0.shape
    BLK = 2048
    G = npad // BLK

    def body(p0_ref, p1_ref, dn_ref, bg_ref, wl_ref, bl_ref, out_ref, acc_ref):
        i = pl.program_id(0)
        inv = 1.0 / (jnp.sum(dn_ref[...], axis=0) + 1e-16)
        o = (p0_ref[...] + p1_ref[...]) * inv[:, None] + bg_ref[...][None, :]
        o = jnp.maximum(o, 0.0)
        rows = lax.broadcasted_iota(i32, (BLK, 1), 0) + i * BLK
        o = jnp.where(rows < n_real, o, 0.0)
        s = jnp.sum(o, axis=0, keepdims=True)

        @pl.when(i == 0)
        def _():
            acc_ref[...] = jnp.zeros((1, D), f32)

        acc_ref[...] = acc_ref[...] + s

        @pl.when(i == G - 1)
        def _():
            pooled = acc_ref[...] * (1.0 / n_real)
            out_ref[...] = (
                jnp.dot(pooled, wl_ref[...], preferred_element_type=f32)
                + bl_ref[...][None, :]
            )

    return pl.pallas_call(
        body,
        grid=(G,),
        in_specs=[
            pl.BlockSpec((BLK, D), lambda i: (i, 0)),
            pl.BlockSpec((BLK, D), lambda i: (i, 0)),
            pl.BlockSpec((NW, BLK), lambda i: (0, i)),
            pl.BlockSpec((D,), lambda i: (0,)),
            pl.BlockSpec((D, D), lambda i: (0, 0)),
            pl.BlockSpec((D,), lambda i: (0,)),
        ],
        out_specs=pl.BlockSpec((1, D), lambda i: (0, 0)),
        out_shape=jax.ShapeDtypeStruct((1, D), f32),
        scratch_shapes=[pltpu.VMEM((1, D), f32)],
    )(p0, p1, denp, bg1, Wl1, bl1)


def _tc_out(pooled, Wout, bout2):
    D, V = Wout.shape
    BV = 8192
    G = pl.cdiv(V, BV)

    def body(p_ref, w_ref, b_ref, o_ref):
        o_ref[...] = (
            jnp.dot(p_ref[...], w_ref[...], preferred_element_type=f32) + b_ref[...]
        )

    return pl.pallas_call(
        body,
        grid=(G,),
        in_specs=[
            pl.BlockSpec((1, D), lambda i: (0, 0)),
            pl.BlockSpec((D, BV), lambda i: (0, i)),
            pl.BlockSpec((1, BV), lambda i: (0, i)),
        ],
        out_specs=pl.BlockSpec((1, BV), lambda i: (0, i)),
        out_shape=jax.ShapeDtypeStruct((1, V), f32),
    )(pooled, Wout, bout2)


def kernel(x, edge_index, emb, Wg0, as0, ad0, bg0, Wl0, bl0,
           Wg1, as1, ad1, bg1, Wl1, bl1, Wout, bout):
    N = x.shape[0]
    V, D = emb.shape
    E = edge_index.shape[1]

    npad = ((N + 1 + 2047) // 2048) * 2048
    et = E + N
    ept = ((et + NW - 1) // NW + 383) // 384 * 384
    etpad = ept * NW

    x_pad = jnp.concatenate([x.astype(i32), jnp.zeros((npad - N,), i32)])
    loops = jnp.arange(N, dtype=i32)
    padi = jnp.full((etpad - et,), N, i32)
    src = jnp.concatenate([edge_index[0].astype(i32), loops, padi])
    dst = jnp.concatenate([edge_index[1].astype(i32), loops, padi])
    dstb = dst.reshape(etpad // 1152, 18, 64)
    bout2 = bout.reshape(1, V)

    h = _emb_gather(x_pad, emb, npad)

    hp0, al0, ar0, c0 = _tc_proj(h, Wg0, as0, ad0)
    ex0, denp0 = _sc_edge_softmax_denom(src, dst, al0, ar0, c0, ept, npad)
    p00, p01 = _sc_aggregate(src, dstb, ex0, hp0, etpad, npad, _K2_NCA, _K2_NCB)

    hp1, al1, ar1, c1 = _tc_mid(p00, p01, denp0, bg0, Wl0, bl0, Wg1, as1, ad1)
    ex1, denp1 = _sc_edge_softmax_denom(src, dst, al1, ar1, c1, ept, npad)
    p10, p11 = _sc_aggregate(src, dstb, ex1, hp1, etpad, npad, _K2_NCA, _K2_NCB)

    pooled = _tc_pool(p10, p11, denp1, bg1, Wl1, bl1, N)
    return _tc_out(pooled, Wout, bout2)
